# Initial kernel scaffold; baseline (speedup 1.0000x reference)
import jax
import jax.numpy as jnp
from jax.experimental import pallas as pl


def kernel(x, W, bias):
    raise NotImplementedError("write your pallas kernel here")


# speedup vs baseline: 1.2573x; 1.2573x over previous
import functools

import jax
import jax.numpy as jnp
from jax import lax
from jax.experimental import pallas as pl
from jax.experimental.pallas import tpu as pltpu
from jax.experimental.pallas import tpu_sc as plsc

_FIELD_DIM = 100000
_NUM_FIELDS = 26
_BATCH = 16384
_LANES = 16
_NUM_WORKERS = 32
_B_PER_W = _BATCH // _NUM_WORKERS
_VECS = _B_PER_W // _LANES


def _sc_body(xt_hbm, table_hbm, bias_hbm, out_hbm, x_v, idx_v, rows_v,
             out_v, bias_v, sem):
    core = lax.axis_index("c")
    sub = lax.axis_index("s")
    wid = sub * 2 + core
    base = wid * _B_PER_W

    pltpu.sync_copy(xt_hbm.at[:, pl.ds(base, _B_PER_W)], x_v)
    pltpu.sync_copy(bias_hbm, bias_v)

    def build(i, _):
        for f in range(_NUM_FIELDS):
            v = x_v[f, pl.ds(i * _LANES, _LANES)]
            idx_v[pl.ds(f * _B_PER_W + i * _LANES, _LANES)] = v + (
                f * _FIELD_DIM)
        return _

    lax.fori_loop(0, _VECS, build, None)

    pltpu.async_copy(table_hbm.at[idx_v], rows_v, sem).wait()

    bias_vec = bias_v[...]

    def reduce(i, _):
        acc = bias_vec
        for f in range(_NUM_FIELDS):
            acc = acc + rows_v[pl.ds(f * _B_PER_W + i * _LANES, _LANES)]
        out_v[pl.ds(i * _LANES, _LANES)] = acc
        return _

    lax.fori_loop(0, _VECS, reduce, None)

    pltpu.sync_copy(out_v, out_hbm.at[pl.ds(base, _B_PER_W)])


@functools.partial(jax.jit, static_argnames=())
def kernel(x, W, bias):
    xt = x.T
    table = W.reshape(-1)
    bias16 = jnp.broadcast_to(bias, (_LANES,))

    mesh = plsc.VectorSubcoreMesh(core_axis_name="c", subcore_axis_name="s")
    run = pl.kernel(
        _sc_body,
        out_type=jax.ShapeDtypeStruct((_BATCH,), jnp.float32),
        mesh=mesh,
        scratch_types=[
            pltpu.VMEM((_NUM_FIELDS, _B_PER_W), jnp.int32),
            pltpu.VMEM((_NUM_FIELDS * _B_PER_W,), jnp.int32),
            pltpu.VMEM((_NUM_FIELDS * _B_PER_W,), jnp.float32),
            pltpu.VMEM((_B_PER_W,), jnp.float32),
            pltpu.VMEM((_LANES,), jnp.float32),
            pltpu.SemaphoreType.DMA,
        ],
    )
    out = run(xt, table, bias16)
    return out.reshape(_BATCH, 1)


# speedup vs baseline: 1.2714x; 1.0112x over previous
import functools

import jax
import jax.numpy as jnp
from jax import lax
from jax.experimental import pallas as pl
from jax.experimental.pallas import tpu as pltpu
from jax.experimental.pallas import tpu_sc as plsc

_FIELD_DIM = 100000
_NUM_FIELDS = 26
_TOTAL_ROWS = _FIELD_DIM * _NUM_FIELDS
_BATCH = 16384
_LANES = 16
_NUM_WORKERS = 32
_B_PER_W = _BATCH // _NUM_WORKERS
_VECS = _B_PER_W // _LANES


def _sc_body(xt_hbm, table_hbm, bias_hbm, out_hbm, x_v, idx_v,
             rows_v, out_v, bias_v, sem):
    core = lax.axis_index("c")
    sub = lax.axis_index("s")
    wid = sub * 2 + core
    base = wid * _B_PER_W

    pltpu.sync_copy(xt_hbm.at[:, pl.ds(base, _B_PER_W)], x_v)
    pltpu.sync_copy(bias_hbm, bias_v)

    def build(i, _):
        for f in range(_NUM_FIELDS):
            v = x_v[f, pl.ds(i * _LANES, _LANES)]
            idx_v[pl.ds(f * _B_PER_W + i * _LANES, _LANES)] = v + (
                f * _FIELD_DIM)
        return _

    lax.fori_loop(0, _VECS, build, None)

    flat_table = table_hbm.at[0]
    pltpu.async_copy(flat_table.at[idx_v], rows_v, sem).wait()

    bias_vec = bias_v[...]

    def reduce(i, _):
        acc = bias_vec
        for f in range(_NUM_FIELDS):
            acc = acc + rows_v[pl.ds(f * _B_PER_W + i * _LANES, _LANES)]
        out_v[pl.ds(i * _LANES, _LANES)] = acc
        return _

    lax.fori_loop(0, _VECS, reduce, None)

    pltpu.sync_copy(out_v, out_hbm.at[pl.ds(base, _B_PER_W)])


@functools.partial(jax.jit, static_argnames=())
def kernel(x, W, bias):
    xt = x.T
    wt = W.T
    bias16 = jnp.broadcast_to(bias, (_LANES,))

    mesh = plsc.VectorSubcoreMesh(core_axis_name="c", subcore_axis_name="s")
    run = pl.kernel(
        _sc_body,
        out_type=jax.ShapeDtypeStruct((_BATCH,), jnp.float32),
        mesh=mesh,
        scratch_types=[
            pltpu.VMEM((_NUM_FIELDS, _B_PER_W), jnp.int32),
            pltpu.VMEM((_NUM_FIELDS * _B_PER_W,), jnp.int32),
            pltpu.VMEM((_NUM_FIELDS * _B_PER_W,), jnp.float32),
            pltpu.VMEM((_B_PER_W,), jnp.float32),
            pltpu.VMEM((_LANES,), jnp.float32),
            pltpu.SemaphoreType.DMA,
        ],
        compiler_params=pltpu.CompilerParams(use_tc_tiling_on_sc=False),
    )
    out = run(xt, wt, bias16)
    return out.reshape(_BATCH, 1)
